# Initial kernel scaffold; baseline (speedup 1.0000x reference)
import jax
import jax.numpy as jnp
from jax.experimental import pallas as pl


def kernel(x_point, x_face, x_edge, ei_pp, ei_fp, ei_ep, ei_pf, ei_ef, ei_ff, ei_pe, ei_fe, batch_point, batch_face, batch_edge, global_features, point_enc, edge_enc, face_enc, convs, norms, gmlp, dec):
    raise NotImplementedError("write your pallas kernel here")


# speedup vs baseline: 1.0865x; 1.0865x over previous
import functools

import jax
import jax.numpy as jnp
from jax import lax
from jax.experimental import pallas as pl
from jax.experimental.pallas import tpu as pltpu
from jax.experimental.pallas import tpu_sc as plsc

HH = 128
NN = 50000
CW = 32
NCH = HH // CW
SC_CORES, SC_TILES = 2, 16
ROWS_PER_TILE = NN // SC_TILES
EP = 524288
GROUP = 512
STREAMS = GROUP // 128
GROUPS_PER_TILE = EP // SC_TILES // GROUP
IDX_ROWS_PER_TILE = EP // SC_TILES // 128

_MESH = plsc.VectorSubcoreMesh(
    core_axis_name="c", subcore_axis_name="s",
    num_cores=SC_CORES, num_subcores=SC_TILES)


@functools.partial(
    pl.kernel,
    out_type=jax.ShapeDtypeStruct((NN, NCH, CW), jnp.float32),
    mesh=_MESH,
    scratch_types=[
        pltpu.VMEM_SHARED((NN + 8, CW), jnp.float32),
        pltpu.VMEM((GROUP, CW), jnp.float32),
        pltpu.VMEM((STREAMS, 128), jnp.int32),
        pltpu.VMEM((STREAMS, 128), jnp.int32),
        pltpu.SemaphoreType.DMA,
        pltpu.SemaphoreType.DMA,
    ],
    compiler_params=pltpu.CompilerParams(use_tc_tiling_on_sc=False),
)
def _sc_segsum(table, srcr, dstr, zeros, out, accum, rows, src_i, dst_i,
               sem_g, sem_s):
    c = lax.axis_index("c")
    s = lax.axis_index("s")
    for p in range(2):
        k = c * 2 + p
        pltpu.sync_copy(zeros, accum.at[pl.ds(s * ROWS_PER_TILE, ROWS_PER_TILE), :])
        plsc.subcore_barrier()

        def group_body(g, carry):
            row0 = s * IDX_ROWS_PER_TILE + g * STREAMS
            pltpu.sync_copy(srcr.at[pl.ds(row0, STREAMS), :], src_i)
            pltpu.sync_copy(dstr.at[pl.ds(row0, STREAMS), :], dst_i)
            for j in range(STREAMS):
                for v in range(8):
                    sl = src_i[j, pl.ds(v * 16, 16)]
                    src_i[j, pl.ds(v * 16, 16)] = sl * NCH + k
            gs = [pltpu.async_copy(table.at[src_i.at[j]],
                                   rows.at[pl.ds(j * 128, 128), :], sem_g)
                  for j in range(STREAMS)]
            for cp in gs:
                cp.wait()
            ss = [pltpu.async_copy(rows.at[pl.ds(j * 128, 128), :],
                                   accum.at[dst_i.at[j]], sem_s, add=True)
                  for j in range(STREAMS)]
            for cp in ss:
                cp.wait()
            return carry

        lax.fori_loop(0, GROUPS_PER_TILE, group_body, 0)
        plsc.subcore_barrier()
        pltpu.sync_copy(accum.at[pl.ds(s * ROWS_PER_TILE, ROWS_PER_TILE), :],
                        out.at[pl.ds(s * ROWS_PER_TILE, ROWS_PER_TILE), k, :])


@functools.partial(
    pl.kernel,
    out_type=jax.ShapeDtypeStruct((SC_CORES, NN, 16), jnp.float32),
    mesh=_MESH,
    scratch_types=[
        pltpu.VMEM_SHARED((NN + 8, 16), jnp.float32),
        pltpu.VMEM((128, 16), jnp.float32),
        pltpu.VMEM((16, 128), jnp.int32),
        pltpu.SemaphoreType.DMA,
    ],
    compiler_params=pltpu.CompilerParams(use_tc_tiling_on_sc=False),
)
def _sc_count(dstr, zeros16, out, accum, ones_v, dst_i, sem_s):
    c = lax.axis_index("c")
    s = lax.axis_index("s")

    def fill(i, carry):
        ones_v[i, :] = jnp.ones((16,), jnp.float32)
        return carry

    lax.fori_loop(0, 128, fill, 0)
    pltpu.sync_copy(zeros16, accum.at[pl.ds(s * ROWS_PER_TILE, ROWS_PER_TILE), :])
    plsc.subcore_barrier()

    def group_body(g, carry):
        row0 = (c * SC_TILES + s) * 128 + g * 16
        pltpu.sync_copy(dstr.at[pl.ds(row0, 16), :], dst_i)
        ss = [pltpu.async_copy(ones_v, accum.at[dst_i.at[j]], sem_s, add=True)
              for j in range(16)]
        for cp in ss:
            cp.wait()
        return carry

    lax.fori_loop(0, 8, group_body, 0)
    plsc.subcore_barrier()
    pltpu.sync_copy(accum.at[pl.ds(s * ROWS_PER_TILE, ROWS_PER_TILE), :],
                    out.at[c, pl.ds(s * ROWS_PER_TILE, ROWS_PER_TILE), :])


_BLK = 2000
_GRID = NN // _BLK


def _encoder(x, params):
    (w1, b1), (w2, b2) = params
    fin = x.shape[1]

    def body(x_ref, w1_ref, b1_ref, w2_ref, b2_ref, out_ref):
        y = jnp.dot(x_ref[...], w1_ref[...], preferred_element_type=jnp.float32, precision=lax.Precision.HIGHEST)
        y = jnp.maximum(y + b1_ref[...], 0.0)
        z = jnp.dot(y, w2_ref[...], preferred_element_type=jnp.float32, precision=lax.Precision.HIGHEST)
        out_ref[...] = z + b2_ref[...]

    return pl.pallas_call(
        body,
        grid=(_GRID,),
        in_specs=[
            pl.BlockSpec((_BLK, fin), lambda i: (i, 0)),
            pl.BlockSpec((fin, HH), lambda i: (0, 0)),
            pl.BlockSpec((1, HH), lambda i: (0, 0)),
            pl.BlockSpec((HH, HH), lambda i: (0, 0)),
            pl.BlockSpec((1, HH), lambda i: (0, 0)),
        ],
        out_specs=pl.BlockSpec((_BLK, HH), lambda i: (i, 0)),
        out_shape=jax.ShapeDtypeStruct((NN, HH), jnp.float32),
    )(x, w1, b1.reshape(1, HH), w2, b2.reshape(1, HH))


def _tc_layer(h, s_list, cc_list, params_list, gamma, beta):
    nrel = len(s_list)

    def body(*refs):
        h_ref = refs[0]
        s_refs = refs[1:1 + nrel]
        cc_refs = refs[1 + nrel:1 + 2 * nrel]
        w_refs = refs[1 + 2 * nrel:1 + 2 * nrel + 3 * nrel]
        g_ref, b_ref = refs[1 + 5 * nrel], refs[2 + 5 * nrel]
        out_ref = refs[3 + 5 * nrel]
        hx = h_ref[...]
        wr_sum = jnp.zeros((HH, HH), jnp.float32)
        o = jnp.zeros((_BLK, HH), jnp.float32)
        for r in range(nrel):
            wl, bl, wr = w_refs[3 * r], w_refs[3 * r + 1], w_refs[3 * r + 2]
            cnt = cc_refs[r][0, :, 0:1] + cc_refs[r][1, :, 0:1]
            inv = 1.0 / jnp.maximum(cnt, 1.0)
            m = s_refs[r][...] * inv
            o = o + jnp.dot(m, wl[...], preferred_element_type=jnp.float32, precision=lax.Precision.HIGHEST) + bl[...]
            wr_sum = wr_sum + wr[...]
        o = o + jnp.dot(hx, wr_sum, preferred_element_type=jnp.float32, precision=lax.Precision.HIGHEST)
        z = hx + jnp.maximum(o, 0.0)
        mu = jnp.mean(z, axis=-1, keepdims=True)
        zc = z - mu
        var = jnp.mean(zc * zc, axis=-1, keepdims=True)
        out_ref[...] = zc * lax.rsqrt(var + 1e-5) * g_ref[...] + b_ref[...]

    in_specs = [pl.BlockSpec((_BLK, HH), lambda i: (i, 0))]
    in_specs += [pl.BlockSpec((_BLK, HH), lambda i: (i, 0))] * nrel
    in_specs += [pl.BlockSpec((SC_CORES, _BLK, 16), lambda i: (0, i, 0))] * nrel
    in_specs += [pl.BlockSpec((HH, HH), lambda i: (0, 0)),
                 pl.BlockSpec((1, HH), lambda i: (0, 0)),
                 pl.BlockSpec((HH, HH), lambda i: (0, 0))] * nrel
    in_specs += [pl.BlockSpec((1, HH), lambda i: (0, 0))] * 2

    args = [h] + list(s_list) + list(cc_list)
    for (wl, bl, wr) in params_list:
        args += [wl, bl.reshape(1, HH), wr]
    args += [gamma.reshape(1, HH), beta.reshape(1, HH)]

    return pl.pallas_call(
        body,
        grid=(_GRID,),
        in_specs=in_specs,
        out_specs=pl.BlockSpec((_BLK, HH), lambda i: (i, 0)),
        out_shape=jax.ShapeDtypeStruct((NN, HH), jnp.float32),
    )(*args)


def _tc_pool(hp, he, hf):
    def body(hp_ref, he_ref, hf_ref, out_ref):
        i = pl.program_id(0)

        @pl.when(i == 0)
        def _():
            out_ref[...] = jnp.zeros((8, HH), jnp.float32)

        sp = jnp.sum(hp_ref[...], axis=0, keepdims=True)
        se = jnp.sum(he_ref[...], axis=0, keepdims=True)
        sf = jnp.sum(hf_ref[...], axis=0, keepdims=True)
        add = jnp.concatenate([sp, se, sf, jnp.zeros((5, HH), jnp.float32)], axis=0)
        out_ref[...] = out_ref[...] + add

    return pl.pallas_call(
        body,
        grid=(_GRID,),
        in_specs=[pl.BlockSpec((_BLK, HH), lambda i: (i, 0))] * 3,
        out_specs=pl.BlockSpec((8, HH), lambda i: (0, 0)),
        out_shape=jax.ShapeDtypeStruct((8, HH), jnp.float32),
    )(hp, he, hf)


def _tc_head(sums, gfeat, gmlp, dec):
    def body(s_ref, g_ref, gw1, gb1, gw2, gb2, gw3, gb3,
             dw1, db1, dw2, db2, dw3, db3, out_ref):
        pools = s_ref[0:3, :] * (1.0 / NN)
        g_gnn = pools.reshape(1, 3 * HH)
        y = jnp.maximum(jnp.dot(g_ref[...], gw1[...],
                                preferred_element_type=jnp.float32, precision=lax.Precision.HIGHEST) + gb1[...], 0.0)
        y = jnp.maximum(jnp.dot(y, gw2[...],
                                preferred_element_type=jnp.float32, precision=lax.Precision.HIGHEST) + gb2[...], 0.0)
        g_phys = jnp.dot(y, gw3[...], preferred_element_type=jnp.float32, precision=lax.Precision.HIGHEST) + gb3[...]
        g_fused = jnp.concatenate([g_gnn, g_phys], axis=1)
        z = jnp.maximum(jnp.dot(g_fused, dw1[...],
                                preferred_element_type=jnp.float32, precision=lax.Precision.HIGHEST) + db1[...], 0.0)
        z = jnp.maximum(jnp.dot(z, dw2[...],
                                preferred_element_type=jnp.float32, precision=lax.Precision.HIGHEST) + db2[...], 0.0)
        out_ref[...] = jnp.dot(z, dw3[...],
                               preferred_element_type=jnp.float32, precision=lax.Precision.HIGHEST) + db3[...]

    (gw1, gb1), (gw2, gb2), (gw3, gb3) = gmlp
    (dw1, db1), (dw2, db2), (dw3, db3) = dec
    args = [sums, gfeat,
            gw1, gb1.reshape(1, -1), gw2, gb2.reshape(1, -1), gw3, gb3.reshape(1, -1),
            dw1, db1.reshape(1, -1), dw2, db2.reshape(1, -1), dw3, db3.reshape(1, -1)]
    in_specs = [pl.BlockSpec(a.shape, lambda i: tuple(0 for _ in a.shape))
                for a in args]
    return pl.pallas_call(
        body,
        grid=(1,),
        in_specs=in_specs,
        out_specs=pl.BlockSpec((1, 1), lambda i: (0, 0)),
        out_shape=jax.ShapeDtypeStruct((1, 1), jnp.float32),
    )(*args)


def _pad_idx(ei):
    e = ei.shape[1]
    pad = EP - e
    srcp = jnp.concatenate([ei[0], jnp.zeros((pad,), jnp.int32)]).reshape(EP // 128, 128)
    dstp = jnp.concatenate([ei[1], jnp.full((pad,), NN, jnp.int32)]).reshape(EP // 128, 128)
    return srcp, dstp


def kernel(x_point, x_face, x_edge, ei_pp, ei_fp, ei_ep, ei_pf, ei_ef, ei_ff,
           ei_pe, ei_fe, batch_point, batch_face, batch_edge, global_features,
           point_enc, edge_enc, face_enc, convs, norms, gmlp, dec):
    rels = {
        "pp": ("p", ei_pp), "fp": ("f", ei_fp), "ep": ("e", ei_ep),
        "pf": ("p", ei_pf), "ef": ("e", ei_ef), "ff": ("f", ei_ff),
        "pe": ("p", ei_pe), "fe": ("f", ei_fe),
    }
    idx = {name: _pad_idx(ei) for name, (_, ei) in rels.items()}
    zeros32 = jnp.zeros((ROWS_PER_TILE, CW), jnp.float32)
    zeros16 = jnp.zeros((ROWS_PER_TILE, 16), jnp.float32)

    hp = _encoder(x_point, point_enc)
    he = _encoder(x_edge, edge_enc)
    hf = _encoder(x_face, face_enc)

    counts = {name: _sc_count(idx[name][1], zeros16) for name in rels}

    for l in range(4):
        L = convs[l]
        tabs = {
            "p": hp.reshape(NN * NCH, CW),
            "f": hf.reshape(NN * NCH, CW),
            "e": he.reshape(NN * NCH, CW),
        }
        seg = {}
        for name, (srct, _) in rels.items():
            srcp, dstp = idx[name]
            seg[name] = _sc_segsum(tabs[srct], srcp, dstp, zeros32).reshape(NN, HH)
        (gp, bp), (ge, be), (gf, bf) = norms[l]
        hp_new = _tc_layer(hp, [seg["pp"], seg["fp"], seg["ep"]],
                           [counts["pp"], counts["fp"], counts["ep"]],
                           [L[0], L[1], L[2]], gp, bp)
        hf_new = _tc_layer(hf, [seg["pf"], seg["ef"], seg["ff"]],
                           [counts["pf"], counts["ef"], counts["ff"]],
                           [L[3], L[4], L[5]], gf, bf)
        he_new = _tc_layer(he, [seg["pe"], seg["fe"]],
                           [counts["pe"], counts["fe"]],
                           [L[6], L[7]], ge, be)
        hp, hf, he = hp_new, hf_new, he_new

    sums = _tc_pool(hp, he, hf)
    return _tc_head(sums, global_features, gmlp, dec)
